# baseline (speedup 1.0000x reference)
import functools

import jax
import jax.numpy as jnp
from jax import lax
from jax.experimental import pallas as pl
from jax.experimental.pallas import tpu as pltpu
from jax.experimental.pallas import tpu_sc as plsc

N_NODES = 10000
N_EDGES = 320000
NC = 2
NS = 16
L = 16

K_BATCH = 96
E_PAD = NC * NS * K_BATCH * 6 * (-(-N_EDGES // (NC * NS * K_BATCH * 6)))
NP = 10752
ROWS_PT = NP // NS


def _mm_body(x_ref, w_ref, o_ref):
    o_ref[...] = jnp.dot(x_ref[...], w_ref[...],
                         preferred_element_type=jnp.float32)


def _mm1(x, W):
    M, K = x.shape
    _, C = W.shape
    blk = 2000
    return pl.pallas_call(
        _mm_body,
        grid=(M // blk,),
        in_specs=[pl.BlockSpec((blk, K), lambda i: (i, 0)),
                  pl.BlockSpec((K, C), lambda i: (0, 0))],
        out_specs=pl.BlockSpec((blk, C), lambda i: (i, 0)),
        out_shape=jax.ShapeDtypeStruct((M, C), jnp.float32),
    )(x, W)


def _mm2_body(a_ref, b_ref, w_ref, o_ref):
    h0 = jax.nn.relu(a_ref[0] + b_ref[0])
    h1 = jax.nn.relu(a_ref[1] + b_ref[1])
    o_ref[...] = (jnp.dot(h0, w_ref[0], preferred_element_type=jnp.float32) +
                  jnp.dot(h1, w_ref[1], preferred_element_type=jnp.float32))


def _mm2(agg, b, W):
    _, _, Cc = agg.shape
    _, _, C = W.shape
    M = N_NODES
    blk = 2000
    return pl.pallas_call(
        _mm2_body,
        grid=(M // blk,),
        in_specs=[pl.BlockSpec((2, blk, Cc), lambda i: (0, i, 0)),
                  pl.BlockSpec((2, Cc), lambda i: (0, 0)),
                  pl.BlockSpec((2, Cc, C), lambda i: (0, 0, 0))],
        out_specs=pl.BlockSpec((blk, C), lambda i: (i, 0)),
        out_shape=jax.ShapeDtypeStruct((M, C), jnp.float32),
    )(agg, b, W)


def _final_body(a_ref, b_ref, o_ref):
    o_ref[...] = jax.nn.relu(a_ref[0] + a_ref[1] + b_ref[...])


def _final(agg, b):
    _, _, C = agg.shape
    M = N_NODES
    blk = 2000
    return pl.pallas_call(
        _final_body,
        grid=(M // blk,),
        in_specs=[pl.BlockSpec((2, blk, C), lambda i: (0, i, 0)),
                  pl.BlockSpec((1, C), lambda i: (0, 0))],
        out_specs=pl.BlockSpec((blk, C), lambda i: (i, 0)),
        out_shape=jax.ShapeDtypeStruct((M, C), jnp.float32),
    )(agg, b)


def _make_spmm(chunked):
    Cc = 128
    K = K_BATCH
    n_workers = NS if chunked else NC * NS
    ept = E_PAD // n_workers
    nb = ept // K
    ni = nb // 6
    mesh = plsc.VectorSubcoreMesh(core_axis_name="c", subcore_axis_name="s")

    @functools.partial(
        pl.kernel,
        out_type=jax.ShapeDtypeStruct((NC, NP, Cc), jnp.float32),
        mesh=mesh,
        scratch_types=[
            pltpu.VMEM((6, K), jnp.int32),
            pltpu.VMEM((6, K), jnp.int32),
            pltpu.VMEM((6, K), jnp.float32),
            pltpu.VMEM((3, K, Cc), jnp.float32),
            pltpu.VMEM_SHARED((NP, Cc), jnp.float32),
            pltpu.SemaphoreType.DMA,
            pltpu.SemaphoreType.DMA,
            pltpu.SemaphoreType.DMA,
            pltpu.SemaphoreType.DMA,
            pltpu.SemaphoreType.DMA,
            pltpu.SemaphoreType.DMA,
            pltpu.SemaphoreType.DMA,
            pltpu.SemaphoreType.DMA,
            pltpu.SemaphoreType.DMA,
        ],
    )
    def spmm(table, rowi, coli, ew, out, colb, rowb, wb, rowsb, acc,
             g0, g1, g2, s0, s1, s2, e0, e1, e2):
        c = lax.axis_index("c")
        s = lax.axis_index("s")
        gsem = (g0, g1, g2)
        ssem = (s0, s1, s2)
        esem = (e0, e1, e2)
        if chunked:
            tile_base = s * ept
        else:
            tile_base = (c * NS + s) * ept

        def ef(b, q):
            base = tile_base + b * K
            pltpu.async_copy(coli.at[pl.ds(base, K)], colb.at[q],
                             esem[q % 3])
            pltpu.async_copy(rowi.at[pl.ds(base, K)], rowb.at[q],
                             esem[q % 3])
            pltpu.async_copy(ew.at[pl.ds(base, K)], wb.at[q], esem[q % 3])

        def ewait(b, q):
            base = tile_base + b * K
            pltpu.make_async_copy(coli.at[pl.ds(base, K)], colb.at[q],
                                  esem[q % 3]).wait()
            pltpu.make_async_copy(rowi.at[pl.ds(base, K)], rowb.at[q],
                                  esem[q % 3]).wait()
            pltpu.make_async_copy(ew.at[pl.ds(base, K)], wb.at[q],
                                  esem[q % 3]).wait()

        def prep(q):
            if chunked:
                for j in range(K // L):
                    sl = pl.ds(j * L, L)
                    colb[q, sl] = colb[q, sl] * 2 + c

        def gfire(q, p):
            pltpu.async_copy(table.at[colb.at[q]], rowsb.at[p], gsem[p])

        def gwait(q, p):
            pltpu.make_async_copy(table.at[colb.at[q]], rowsb.at[p],
                                  gsem[p]).wait()

        def sfire(q, p):
            pltpu.async_copy(rowsb.at[p], acc.at[rowb.at[q]], ssem[p],
                             add=True)

        def swait(q, p):
            pltpu.make_async_copy(rowsb.at[p], acc.at[rowb.at[q]],
                                  ssem[p]).wait()

        def scale(q, p):
            def sbody(j, inner):
                w16 = wb[q, pl.ds(j * L, L)]
                for jj in range(L):
                    e = j * L + jj
                    wv = jnp.broadcast_to(w16[jj], (L,))
                    for cc in range(Cc // L):
                        sl = pl.ds(cc * L, L)
                        rowsb[p, e, sl] = rowsb[p, e, sl] * wv
                return inner

            lax.fori_loop(0, K // L, sbody, 0)

        def consume(b, j, do_gather, do_fetch):
            gwait(j, j % 3)
            scale(j, j % 3)
            sfire(j, j % 3)
            swait((j - 1) % 6, (j - 1) % 3)
            if do_gather:
                ewait(b + 2, (j + 2) % 6)
                prep((j + 2) % 6)
                gfire((j + 2) % 6, (j + 2) % 3)
            if do_fetch:
                ef(b + 3, (j + 3) % 6)

        zero = jnp.zeros((L,), jnp.float32)
        izero = jnp.zeros((L,), jnp.int32)
        for j in range(K // L):
            rowb[5, pl.ds(j * L, L)] = izero

        def zfill(e, carry):
            for cc in range(Cc // L):
                rowsb[2, e, pl.ds(cc * L, L)] = zero
            return carry

        lax.fori_loop(0, K, zfill, 0)
        for kk in range(ROWS_PT // K):
            pltpu.sync_copy(rowsb.at[2],
                            acc.at[pl.ds(s * ROWS_PT + kk * K, K)])
        plsc.subcore_barrier()

        ef(0, 0)
        ef(1, 1)
        ef(2, 2)
        ewait(0, 0)
        prep(0)
        gfire(0, 0)
        ewait(1, 1)
        prep(1)
        gfire(1, 1)
        sfire(5, 2)

        def body(i, carry):
            bb = 6 * i
            for j in range(6):
                consume(bb + j, j, True, True)
            return carry

        lax.fori_loop(0, ni - 1, body, 0)

        bb = nb - 6
        for j in range(6):
            consume(bb + j, j, j < 4, j < 3)
        swait(5, 2)

        plsc.subcore_barrier()

        for kk in range(ROWS_PT // K):
            r0 = s * ROWS_PT + kk * K
            pltpu.sync_copy(acc.at[pl.ds(r0, K)], out.at[c, pl.ds(r0, K)])

    return spmm


_spmm_chunked = _make_spmm(True)
_spmm_split = _make_spmm(False)


@jax.jit
def kernel(x, edge_index, edge_weight, W1, b1, W2, b2):
    row = edge_index[0]
    col = edge_index[1]
    pad = E_PAD - N_EDGES
    rowp = jnp.pad(row, (0, pad))
    colp = jnp.pad(col, (0, pad))
    ewp = jnp.pad(edge_weight, (0, pad))

    s1 = _mm1(x, W1)
    agg1 = _spmm_chunked(s1.reshape(2 * N_NODES, 128), rowp, colp, ewp)
    s2 = _mm2(agg1, b1.reshape(2, 128), W2.reshape(2, 128, 128))
    agg2 = _spmm_split(s2, rowp, colp, ewp)
    return _final(agg2, b2.reshape(1, 128))


# speedup vs baseline: 3.8380x; 3.8380x over previous
import functools

import jax
import jax.numpy as jnp
from jax import lax
from jax.experimental import pallas as pl
from jax.experimental.pallas import tpu as pltpu
from jax.experimental.pallas import tpu_sc as plsc

N_NODES = 10000
N_EDGES = 320000
NC = 2
NS = 16
L = 16

K_BATCH = 96
E_PAD = NC * NS * K_BATCH * 6 * (-(-N_EDGES // (NC * NS * K_BATCH * 6)))
NP = 10752
ROWS_PT = NP // NS


def _mm_body(x_ref, w_ref, o_ref):
    o_ref[...] = jnp.dot(x_ref[...], w_ref[...],
                         preferred_element_type=jnp.float32)


def _mm1(x, W):
    M, K = x.shape
    _, C = W.shape
    blk = 2000
    return pl.pallas_call(
        _mm_body,
        grid=(M // blk,),
        in_specs=[pl.BlockSpec((blk, K), lambda i: (i, 0)),
                  pl.BlockSpec((K, C), lambda i: (0, 0))],
        out_specs=pl.BlockSpec((blk, C), lambda i: (i, 0)),
        out_shape=jax.ShapeDtypeStruct((M, C), jnp.float32),
    )(x, W)


def _mm2_body(a_ref, b_ref, w_ref, o_ref):
    h0 = jax.nn.relu(a_ref[0] + b_ref[0])
    h1 = jax.nn.relu(a_ref[1] + b_ref[1])
    o_ref[...] = (jnp.dot(h0, w_ref[0], preferred_element_type=jnp.float32) +
                  jnp.dot(h1, w_ref[1], preferred_element_type=jnp.float32))


def _mm2(agg, b, W):
    _, _, Cc = agg.shape
    _, _, C = W.shape
    M = N_NODES
    blk = 2000
    return pl.pallas_call(
        _mm2_body,
        grid=(M // blk,),
        in_specs=[pl.BlockSpec((2, blk, Cc), lambda i: (0, i, 0)),
                  pl.BlockSpec((2, Cc), lambda i: (0, 0)),
                  pl.BlockSpec((2, Cc, C), lambda i: (0, 0, 0))],
        out_specs=pl.BlockSpec((blk, C), lambda i: (i, 0)),
        out_shape=jax.ShapeDtypeStruct((M, C), jnp.float32),
    )(agg, b, W)


def _final_body(a_ref, b_ref, o_ref):
    o_ref[...] = jax.nn.relu(a_ref[0] + a_ref[1] + b_ref[...])


def _final(agg, b):
    _, _, C = agg.shape
    M = N_NODES
    blk = 2000
    return pl.pallas_call(
        _final_body,
        grid=(M // blk,),
        in_specs=[pl.BlockSpec((2, blk, C), lambda i: (0, i, 0)),
                  pl.BlockSpec((1, C), lambda i: (0, 0))],
        out_specs=pl.BlockSpec((blk, C), lambda i: (i, 0)),
        out_shape=jax.ShapeDtypeStruct((M, C), jnp.float32),
    )(agg, b)


def _make_spmm(chunked):
    Cc = 128
    K = K_BATCH
    n_workers = NS if chunked else NC * NS
    ept = E_PAD // n_workers
    nb = ept // K
    ni = nb // 6
    mesh = plsc.VectorSubcoreMesh(core_axis_name="c", subcore_axis_name="s")

    @functools.partial(
        pl.kernel,
        out_type=jax.ShapeDtypeStruct((NC, NP, Cc), jnp.float32),
        mesh=mesh,
        scratch_types=[
            pltpu.VMEM((6, K), jnp.int32),
            pltpu.VMEM((6, K), jnp.int32),
            pltpu.VMEM((6, K), jnp.float32),
            pltpu.VMEM((3, K, Cc), jnp.float32),
            pltpu.VMEM_SHARED((NP, Cc), jnp.float32),
            pltpu.SemaphoreType.DMA,
            pltpu.SemaphoreType.DMA,
            pltpu.SemaphoreType.DMA,
            pltpu.SemaphoreType.DMA,
            pltpu.SemaphoreType.DMA,
            pltpu.SemaphoreType.DMA,
            pltpu.SemaphoreType.DMA,
            pltpu.SemaphoreType.DMA,
            pltpu.SemaphoreType.DMA,
        ],
    )
    def spmm(table, rowi, coli, ew, out, colb, rowb, wb, rowsb, acc,
             g0, g1, g2, s0, s1, s2, e0, e1, e2):
        c = lax.axis_index("c")
        s = lax.axis_index("s")
        gsem = (g0, g1, g2)
        ssem = (s0, s1, s2)
        esem = (e0, e1, e2)
        if chunked:
            tile_base = s * ept
        else:
            tile_base = (c * NS + s) * ept

        def ef(b, q):
            base = tile_base + b * K
            pltpu.async_copy(coli.at[pl.ds(base, K)], colb.at[q],
                             esem[q % 3])
            pltpu.async_copy(rowi.at[pl.ds(base, K)], rowb.at[q],
                             esem[q % 3])
            pltpu.async_copy(ew.at[pl.ds(base, K)], wb.at[q], esem[q % 3])

        def ewait(b, q):
            base = tile_base + b * K
            pltpu.make_async_copy(coli.at[pl.ds(base, K)], colb.at[q],
                                  esem[q % 3]).wait()
            pltpu.make_async_copy(rowi.at[pl.ds(base, K)], rowb.at[q],
                                  esem[q % 3]).wait()
            pltpu.make_async_copy(ew.at[pl.ds(base, K)], wb.at[q],
                                  esem[q % 3]).wait()

        def prep(q):
            if chunked:
                for j in range(K // L):
                    sl = pl.ds(j * L, L)
                    colb[q, sl] = colb[q, sl] * 2 + c

        def gfire(q, p):
            pltpu.async_copy(table.at[colb.at[q]], rowsb.at[p], gsem[p])

        def gwait(q, p):
            pltpu.make_async_copy(table.at[colb.at[q]], rowsb.at[p],
                                  gsem[p]).wait()

        def sfire(q, p):
            pltpu.async_copy(rowsb.at[p], acc.at[rowb.at[q]], ssem[p],
                             add=True)

        def swait(q, p):
            pltpu.make_async_copy(rowsb.at[p], acc.at[rowb.at[q]],
                                  ssem[p]).wait()

        def scale(q, p):
            def sbody(j, inner):
                w16 = wb[q, pl.ds(j * L, L)]
                for jj in range(L):
                    e = j * L + jj
                    wv = jnp.broadcast_to(w16[jj], (L,))
                    for cc in range(Cc // L):
                        sl = pl.ds(cc * L, L)
                        rowsb[p, e, sl] = rowsb[p, e, sl] * wv
                return inner

            lax.fori_loop(0, K // L, sbody, 0)

        def consume(b, j, do_gather, do_fetch):
            gwait(j, j % 3)
            scale(j, j % 3)
            sfire(j, j % 3)
            swait((j - 1) % 6, (j - 1) % 3)
            if do_gather:
                ewait(b + 2, (j + 2) % 6)
                prep((j + 2) % 6)
                gfire((j + 2) % 6, (j + 2) % 3)
            if do_fetch:
                ef(b + 3, (j + 3) % 6)

        zero = jnp.zeros((L,), jnp.float32)
        izero = jnp.zeros((L,), jnp.int32)
        for j in range(K // L):
            rowb[5, pl.ds(j * L, L)] = izero

        def zfill(e, carry):
            for cc in range(Cc // L):
                rowsb[2, e, pl.ds(cc * L, L)] = zero
            return carry

        lax.fori_loop(0, K, zfill, 0)
        for kk in range(ROWS_PT // K):
            pltpu.sync_copy(rowsb.at[2],
                            acc.at[pl.ds(s * ROWS_PT + kk * K, K)])
        plsc.subcore_barrier()

        ef(0, 0)
        ef(1, 1)
        ef(2, 2)
        ewait(0, 0)
        prep(0)
        gfire(0, 0)
        ewait(1, 1)
        prep(1)
        gfire(1, 1)
        sfire(5, 2)

        def body(i, carry):
            bb = 6 * i
            for j in range(6):
                consume(bb + j, j, True, True)
            return carry

        lax.fori_loop(0, ni - 1, body, 0)

        bb = nb - 6
        for j in range(6):
            consume(bb + j, j, j < 4, j < 3)
        swait(5, 2)

        plsc.subcore_barrier()

        for kk in range(ROWS_PT // K):
            r0 = s * ROWS_PT + kk * K
            pltpu.sync_copy(acc.at[pl.ds(r0, K)], out.at[c, pl.ds(r0, K)])

    return spmm


_spmm_chunked = _make_spmm(True)
_spmm_split = _make_spmm(False)


@jax.jit
def kernel(x, edge_index, edge_weight, W1, b1, W2, b2):
    row = edge_index[0]
    col = edge_index[1]
    pad = E_PAD - N_EDGES
    ar = jnp.arange(pad, dtype=jnp.int32)
    rowp = jnp.concatenate([row, N_NODES + ar % (NP - N_NODES)])
    colp = jnp.concatenate([col, ar % N_NODES])
    ewp = jnp.pad(edge_weight, (0, pad))

    s1 = _mm1(x, W1)
    agg1 = _spmm_chunked(s1.reshape(2 * N_NODES, 128), rowp, colp, ewp)
    s2 = _mm2(agg1, b1.reshape(2, 128), W2.reshape(2, 128, 128))
    agg2 = _spmm_split(s2, rowp, colp, ewp)
    return _final(agg2, b2.reshape(1, 128))


# speedup vs baseline: 5.4713x; 1.4256x over previous
import functools

import jax
import jax.numpy as jnp
from jax import lax
from jax.experimental import pallas as pl
from jax.experimental.pallas import tpu as pltpu
from jax.experimental.pallas import tpu_sc as plsc

N_NODES = 10000
N_EDGES = 320000
NC = 2
NS = 16
L = 16

K_BATCH = 96
E_PAD = NC * NS * K_BATCH * 6 * (-(-N_EDGES // (NC * NS * K_BATCH * 6)))
NP = 10752
ROWS_PT = NP // NS


def _mm12_body(a_ref, w1_ref, b1_ref, w2_ref, o_ref):
    a = a_ref[0] + a_ref[1]
    h = jax.nn.relu(
        jnp.dot(a, w1_ref[...], preferred_element_type=jnp.float32)
        + b1_ref[...])
    o_ref[...] = jnp.dot(h, w2_ref[...], preferred_element_type=jnp.float32)


def _mm12(agg, W1, b1, W2):
    _, _, Cc = agg.shape
    C1 = W1.shape[1]
    C2 = W2.shape[1]
    M = N_NODES
    blk = 2000
    return pl.pallas_call(
        _mm12_body,
        grid=(M // blk,),
        in_specs=[pl.BlockSpec((2, blk, Cc), lambda i: (0, i, 0)),
                  pl.BlockSpec((Cc, C1), lambda i: (0, 0)),
                  pl.BlockSpec((1, C1), lambda i: (0, 0)),
                  pl.BlockSpec((C1, C2), lambda i: (0, 0))],
        out_specs=pl.BlockSpec((blk, C2), lambda i: (i, 0)),
        out_shape=jax.ShapeDtypeStruct((M, C2), jnp.float32),
    )(agg, W1, b1, W2)


def _final_body(a_ref, b_ref, o_ref):
    o_ref[...] = jax.nn.relu(a_ref[0] + a_ref[1] + b_ref[...])


def _final(agg, b):
    _, _, C = agg.shape
    M = N_NODES
    blk = 2000
    return pl.pallas_call(
        _final_body,
        grid=(M // blk,),
        in_specs=[pl.BlockSpec((2, blk, C), lambda i: (0, i, 0)),
                  pl.BlockSpec((1, C), lambda i: (0, 0))],
        out_specs=pl.BlockSpec((blk, C), lambda i: (i, 0)),
        out_shape=jax.ShapeDtypeStruct((M, C), jnp.float32),
    )(agg, b)


def _make_spmm():
    Cc = 128
    K = K_BATCH
    n_workers = NC * NS
    ept = E_PAD // n_workers
    nb = ept // K
    ni = nb // 6
    mesh = plsc.VectorSubcoreMesh(core_axis_name="c", subcore_axis_name="s")

    @functools.partial(
        pl.kernel,
        out_type=jax.ShapeDtypeStruct((NC, NP, Cc), jnp.float32),
        mesh=mesh,
        scratch_types=[
            pltpu.VMEM((6, K), jnp.int32),
            pltpu.VMEM((6, K), jnp.int32),
            pltpu.VMEM((6, K), jnp.float32),
            pltpu.VMEM((3, K, Cc), jnp.float32),
            pltpu.VMEM_SHARED((NP, Cc), jnp.float32),
            pltpu.SemaphoreType.DMA,
            pltpu.SemaphoreType.DMA,
            pltpu.SemaphoreType.DMA,
            pltpu.SemaphoreType.DMA,
            pltpu.SemaphoreType.DMA,
            pltpu.SemaphoreType.DMA,
            pltpu.SemaphoreType.DMA,
            pltpu.SemaphoreType.DMA,
            pltpu.SemaphoreType.DMA,
        ],
    )
    def spmm(table, rowi, coli, ew, out, colb, rowb, wb, rowsb, acc,
             g0, g1, g2, s0, s1, s2, e0, e1, e2):
        c = lax.axis_index("c")
        s = lax.axis_index("s")
        gsem = (g0, g1, g2)
        ssem = (s0, s1, s2)
        esem = (e0, e1, e2)
        tile_base = (c * NS + s) * ept

        def ef(b, q):
            base = tile_base + b * K
            pltpu.async_copy(coli.at[pl.ds(base, K)], colb.at[q],
                             esem[q % 3])
            pltpu.async_copy(rowi.at[pl.ds(base, K)], rowb.at[q],
                             esem[q % 3])
            pltpu.async_copy(ew.at[pl.ds(base, K)], wb.at[q], esem[q % 3])

        def ewait(b, q):
            base = tile_base + b * K
            pltpu.make_async_copy(coli.at[pl.ds(base, K)], colb.at[q],
                                  esem[q % 3]).wait()
            pltpu.make_async_copy(rowi.at[pl.ds(base, K)], rowb.at[q],
                                  esem[q % 3]).wait()
            pltpu.make_async_copy(ew.at[pl.ds(base, K)], wb.at[q],
                                  esem[q % 3]).wait()

        def gfire(q, p):
            pltpu.async_copy(table.at[colb.at[q]], rowsb.at[p], gsem[p])

        def gwait(q, p):
            pltpu.make_async_copy(table.at[colb.at[q]], rowsb.at[p],
                                  gsem[p]).wait()

        def sfire(q, p):
            pltpu.async_copy(rowsb.at[p], acc.at[rowb.at[q]], ssem[p],
                             add=True)

        def swait(q, p):
            pltpu.make_async_copy(rowsb.at[p], acc.at[rowb.at[q]],
                                  ssem[p]).wait()

        def scale(q, p):
            def sbody(j, inner):
                w16 = wb[q, pl.ds(j * L, L)]
                for jj in range(L):
                    e = j * L + jj
                    wv = jnp.broadcast_to(w16[jj], (L,))
                    for cc in range(Cc // L):
                        sl = pl.ds(cc * L, L)
                        rowsb[p, e, sl] = rowsb[p, e, sl] * wv
                return inner

            lax.fori_loop(0, K // L, sbody, 0)

        def consume(b, j, do_gather, do_fetch):
            gwait(j, j % 3)
            scale(j, j % 3)
            sfire(j, j % 3)
            swait((j - 1) % 6, (j - 1) % 3)
            if do_gather:
                ewait(b + 2, (j + 2) % 6)
                gfire((j + 2) % 6, (j + 2) % 3)
            if do_fetch:
                ef(b + 3, (j + 3) % 6)

        zero = jnp.zeros((L,), jnp.float32)
        izero = jnp.zeros((L,), jnp.int32)
        for j in range(K // L):
            rowb[5, pl.ds(j * L, L)] = izero

        def zfill(e, carry):
            for cc in range(Cc // L):
                rowsb[2, e, pl.ds(cc * L, L)] = zero
            return carry

        lax.fori_loop(0, K, zfill, 0)
        for kk in range(ROWS_PT // K):
            pltpu.sync_copy(rowsb.at[2],
                            acc.at[pl.ds(s * ROWS_PT + kk * K, K)])
        plsc.subcore_barrier()

        ef(0, 0)
        ef(1, 1)
        ef(2, 2)
        ewait(0, 0)
        gfire(0, 0)
        ewait(1, 1)
        gfire(1, 1)
        sfire(5, 2)

        def body(i, carry):
            bb = 6 * i
            for j in range(6):
                consume(bb + j, j, True, True)
            return carry

        lax.fori_loop(0, ni - 1, body, 0)

        bb = nb - 6
        for j in range(6):
            consume(bb + j, j, j < 4, j < 3)
        swait(5, 2)

        plsc.subcore_barrier()

        for kk in range(ROWS_PT // K):
            r0 = s * ROWS_PT + kk * K
            pltpu.sync_copy(acc.at[pl.ds(r0, K)], out.at[c, pl.ds(r0, K)])

    return spmm


_spmm = _make_spmm()


@jax.jit
def kernel(x, edge_index, edge_weight, W1, b1, W2, b2):
    row = edge_index[0]
    col = edge_index[1]
    pad = E_PAD - N_EDGES
    ar = jnp.arange(pad, dtype=jnp.int32)
    rowp = jnp.concatenate([row, N_NODES + ar % (NP - N_NODES)])
    colp = jnp.concatenate([col, ar % N_NODES])
    ewp = jnp.pad(edge_weight, (0, pad))

    aggx = _spmm(x, rowp, colp, ewp)
    s2 = _mm12(aggx, W1, b1.reshape(1, 256), W2)
    agg2 = _spmm(s2, rowp, colp, ewp)
    return _final(agg2, b2.reshape(1, 128))


# speedup vs baseline: 5.4744x; 1.0006x over previous
import functools

import jax
import jax.numpy as jnp
from jax import lax
from jax.experimental import pallas as pl
from jax.experimental.pallas import tpu as pltpu
from jax.experimental.pallas import tpu_sc as plsc

N_NODES = 10000
N_EDGES = 320000
NC = 2
NS = 16
L = 16

K_BATCH = 96
E_PAD = NC * NS * K_BATCH * 6 * (-(-N_EDGES // (NC * NS * K_BATCH * 6)))
NP = 10752
ROWS_PT = NP // NS


def _mm12_body(a_ref, w1_ref, b1_ref, w2_ref, o_ref):
    a = a_ref[0] + a_ref[1]
    h = jax.nn.relu(
        jnp.dot(a, w1_ref[...], preferred_element_type=jnp.float32)
        + b1_ref[...])
    o_ref[...] = jnp.dot(h, w2_ref[...], preferred_element_type=jnp.float32)


def _mm12(agg, W1, b1, W2):
    _, _, Cc = agg.shape
    C1 = W1.shape[1]
    C2 = W2.shape[1]
    M = N_NODES
    blk = 2000
    return pl.pallas_call(
        _mm12_body,
        grid=(M // blk,),
        in_specs=[pl.BlockSpec((2, blk, Cc), lambda i: (0, i, 0)),
                  pl.BlockSpec((Cc, C1), lambda i: (0, 0)),
                  pl.BlockSpec((1, C1), lambda i: (0, 0)),
                  pl.BlockSpec((C1, C2), lambda i: (0, 0))],
        out_specs=pl.BlockSpec((blk, C2), lambda i: (i, 0)),
        out_shape=jax.ShapeDtypeStruct((M, C2), jnp.float32),
    )(agg, W1, b1, W2)


def _final_body(a_ref, b_ref, o_ref):
    o_ref[...] = jax.nn.relu(a_ref[0] + a_ref[1] + b_ref[...])


def _final(agg, b):
    _, _, C = agg.shape
    M = N_NODES
    blk = 2000
    return pl.pallas_call(
        _final_body,
        grid=(M // blk,),
        in_specs=[pl.BlockSpec((2, blk, C), lambda i: (0, i, 0)),
                  pl.BlockSpec((1, C), lambda i: (0, 0))],
        out_specs=pl.BlockSpec((blk, C), lambda i: (i, 0)),
        out_shape=jax.ShapeDtypeStruct((M, C), jnp.float32),
    )(agg, b)


def _make_spmm():
    Cc = 128
    K = K_BATCH
    n_workers = NC * NS
    ept = E_PAD // n_workers
    nb = ept // K
    ni = nb // 6
    mesh = plsc.VectorSubcoreMesh(core_axis_name="c", subcore_axis_name="s")

    @functools.partial(
        pl.kernel,
        out_type=jax.ShapeDtypeStruct((NC, NP, Cc), jnp.float32),
        mesh=mesh,
        scratch_types=[
            pltpu.VMEM((6, K), jnp.int32),
            pltpu.VMEM((6, K), jnp.int32),
            pltpu.VMEM((3, K), jnp.float32),
            pltpu.VMEM((3, K, Cc), jnp.float32),
            pltpu.VMEM_SHARED((NP, Cc), jnp.float32),
            pltpu.SemaphoreType.DMA,
            pltpu.SemaphoreType.DMA,
            pltpu.SemaphoreType.DMA,
            pltpu.SemaphoreType.DMA,
            pltpu.SemaphoreType.DMA,
            pltpu.SemaphoreType.DMA,
            pltpu.SemaphoreType.DMA,
            pltpu.SemaphoreType.DMA,
            pltpu.SemaphoreType.DMA,
        ],
    )
    def spmm(table, rowi, coli, ew, out, colb, rowb, wb, rowsb, acc,
             g0, g1, g2, s0, s1, s2, e0, e1, e2):
        c = lax.axis_index("c")
        s = lax.axis_index("s")
        gsem = (g0, g1, g2)
        ssem = (s0, s1, s2)
        esem = (e0, e1, e2)
        tile_base = (c * NS + s) * ept

        def ef(b, q):
            base = tile_base + b * K
            pltpu.async_copy(coli.at[pl.ds(base, K)], colb.at[q],
                             esem[q % 3])
            pltpu.async_copy(rowi.at[pl.ds(base, K)], rowb.at[q],
                             esem[q % 3])
            pltpu.async_copy(ew.at[pl.ds(base, K)], wb.at[q % 3],
                             esem[q % 3])

        def ewait(b, q):
            base = tile_base + b * K
            pltpu.make_async_copy(coli.at[pl.ds(base, K)], colb.at[q],
                                  esem[q % 3]).wait()
            pltpu.make_async_copy(rowi.at[pl.ds(base, K)], rowb.at[q],
                                  esem[q % 3]).wait()
            pltpu.make_async_copy(ew.at[pl.ds(base, K)], wb.at[q % 3],
                                  esem[q % 3]).wait()

        def gfire(q, p):
            pltpu.async_copy(table.at[colb.at[q]], rowsb.at[p], gsem[p])

        def gwait(q, p):
            pltpu.make_async_copy(table.at[colb.at[q]], rowsb.at[p],
                                  gsem[p]).wait()

        def sfire(q, p):
            pltpu.async_copy(rowsb.at[p], acc.at[rowb.at[q]], ssem[p],
                             add=True)

        def swait(q, p):
            pltpu.make_async_copy(rowsb.at[p], acc.at[rowb.at[q]],
                                  ssem[p]).wait()

        def scale(p):
            def sbody(j, inner):
                wv16 = wb[p, pl.ds(j * L, L)]
                for jj in range(L):
                    e = j * L + jj
                    wv = jnp.broadcast_to(wv16[jj], (L,))
                    for cc in range(Cc // L):
                        sl = pl.ds(cc * L, L)
                        rowsb[p, e, sl] = rowsb[p, e, sl] * wv
                return inner

            lax.fori_loop(0, K // L, sbody, 0)

        def consume(b, j, do_gather, do_fetch):
            gwait(j, j % 3)
            scale(j % 3)
            sfire(j, j % 3)
            swait((j - 1) % 6, (j - 1) % 3)
            if do_gather:
                ewait(b + 2, (j + 2) % 6)
                gfire((j + 2) % 6, (j + 2) % 3)
            if do_fetch:
                ef(b + 3, (j + 3) % 6)

        zero = jnp.zeros((L,), jnp.float32)
        izero = jnp.zeros((L,), jnp.int32)
        for j in range(K // L):
            rowb[5, pl.ds(j * L, L)] = izero

        def zfill(e, carry):
            for cc in range(Cc // L):
                rowsb[2, e, pl.ds(cc * L, L)] = zero
            return carry

        lax.fori_loop(0, K, zfill, 0)
        for kk in range(ROWS_PT // K):
            pltpu.sync_copy(rowsb.at[2],
                            acc.at[pl.ds(s * ROWS_PT + kk * K, K)])
        plsc.subcore_barrier()

        ef(0, 0)
        ef(1, 1)
        ef(2, 2)
        ewait(0, 0)
        gfire(0, 0)
        ewait(1, 1)
        gfire(1, 1)
        sfire(5, 2)

        def body(i, carry):
            bb = 6 * i
            for j in range(6):
                consume(bb + j, j, True, True)
            return carry

        lax.fori_loop(0, ni - 1, body, 0)

        bb = nb - 6
        for j in range(6):
            consume(bb + j, j, j < 4, j < 3)
        swait(5, 2)

        plsc.subcore_barrier()

        for kk in range(ROWS_PT // K):
            r0 = s * ROWS_PT + kk * K
            pltpu.sync_copy(acc.at[pl.ds(r0, K)], out.at[c, pl.ds(r0, K)])

    return spmm


_spmm = _make_spmm()


@jax.jit
def kernel(x, edge_index, edge_weight, W1, b1, W2, b2):
    row = edge_index[0]
    col = edge_index[1]
    pad = E_PAD - N_EDGES
    ar = jnp.arange(pad, dtype=jnp.int32)
    rowp = jnp.concatenate([row, N_NODES + ar % (NP - N_NODES)])
    colp = jnp.concatenate([col, ar % N_NODES])
    ewp = jnp.pad(edge_weight, (0, pad))

    aggx = _spmm(x, rowp, colp, ewp)
    s2 = _mm12(aggx, W1, b1.reshape(1, 256), W2)
    agg2 = _spmm(s2, rowp, colp, ewp)
    return _final(agg2, b2.reshape(1, 128))
